# baseline (speedup 1.0000x reference)
import functools
import math

import jax
import jax.numpy as jnp
from jax import lax
from jax.experimental import pallas as pl
from jax.experimental.pallas import tpu as pltpu
from jax.experimental.pallas import tpu_sc as plsc

D_MODEL = 64
SCALE = math.sqrt(D_MODEL)
NC = 2
NS = 16
NW = NC * NS
NBUF = 4
LANES = 16


def _make_kernel(bsz: int, seq: int):
    ka = min(128, (seq // 2 + 7) // 8 * 8)
    kb = seq - ka
    assert 0 < kb <= 128 and ka % 8 == 0 and kb % 8 == 0
    assert bsz % NW == 0
    xrows_w = bsz // NW
    chunks_w = 2 * xrows_w
    assert chunks_w % NBUF == 0 and chunks_w // NBUF >= 2
    n_groups = chunks_w // NBUF
    k_of = [ka if b % 2 == 0 else kb for b in range(NBUF)]
    off_of = [0 if b % 2 == 0 else ka for b in range(NBUF)]

    mesh = plsc.VectorSubcoreMesh(core_axis_name="c", subcore_axis_name="s")

    @functools.partial(
        pl.kernel,
        out_type=jax.ShapeDtypeStruct((bsz, seq, D_MODEL), jnp.float32),
        mesh=mesh,
        scratch_types=[
            pltpu.VMEM((xrows_w, seq), jnp.int32),
            pltpu.VMEM((NBUF, ka, D_MODEL), jnp.float32),
            pltpu.VMEM((NBUF, ka, D_MODEL), jnp.float32),
        ]
        + [pltpu.SemaphoreType.DMA] * (2 * NBUF),
        compiler_params=pltpu.CompilerParams(use_tc_tiling_on_sc=False),
    )
    def emb(x_hbm, lut_hbm, out_hbm, idx_v, row_v, sc_v, *sems):
        gsem = sems[:NBUF]
        osem = sems[NBUF:]
        wid = lax.axis_index("s") * NC + lax.axis_index("c")
        xrow0 = wid * xrows_w

        pltpu.sync_copy(x_hbm.at[pl.ds(xrow0, xrows_w)], idx_v)

        def idx_slice(c, b):
            return idx_v.at[c >> 1, pl.ds(off_of[b], k_of[b])]

        def out_slice(c, b):
            return out_hbm.at[xrow0 + (c >> 1), pl.ds(off_of[b], k_of[b])]

        def buf_slice(ring, b):
            return ring.at[b, pl.ds(0, k_of[b])]

        def start_gather(c, b):
            pltpu.async_copy(
                lut_hbm.at[idx_slice(c, b)], buf_slice(row_v, b), gsem[b]
            )

        def wait_gather(c, b):
            pltpu.make_async_copy(
                lut_hbm.at[idx_slice(c, b)], buf_slice(row_v, b), gsem[b]
            ).wait()

        def scale(b):
            src = row_v.at[b]
            dst = sc_v.at[b]

            def body(r, _):
                for j in range(D_MODEL // LANES):
                    sl = pl.ds(j * LANES, LANES)
                    dst[r, sl] = src[r, sl] * SCALE
                return 0

            lax.fori_loop(0, k_of[b], body, 0, unroll=2)

        def start_out(c, b):
            pltpu.async_copy(buf_slice(sc_v, b), out_slice(c, b), osem[b])

        def wait_out(c, b):
            pltpu.make_async_copy(
                buf_slice(sc_v, b), out_slice(c, b), osem[b]
            ).wait()

        for b in range(NBUF):
            start_gather(b, b)

        def group(g, _):
            for b in range(NBUF):
                c = g * NBUF + b
                wait_gather(c, b)

                @pl.when(g > 0)
                def _():
                    wait_out(c - NBUF, b)

                scale(b)
                start_out(c, b)

                @pl.when(g < n_groups - 1)
                def _():
                    start_gather(c + NBUF, b)
            return 0

        lax.fori_loop(0, n_groups, group, 0)

        for b in range(NBUF):
            c = (n_groups - 1) * NBUF + b
            wait_out(c, b)

    return emb


def kernel(x, lut):
    bsz, seq = x.shape
    vocab, d = lut.shape
    assert d == D_MODEL
    return _make_kernel(bsz, seq)(x.astype(jnp.int32), lut)
